# baseline (speedup 1.0000x reference)
import jax
import jax.numpy as jnp
from jax import lax
from jax.experimental import pallas as pl
from jax.experimental.pallas import tpu as pltpu
from jax.experimental.pallas import tpu_sc as plsc

N_USERS = 25000
N_ITEMS = 25000
N_NODES = N_USERS + N_ITEMS
LATENT = 64
N_EDGES = 800000

NC = 2
NS = 16
HALF = LATENT // NC

CHUNK = 128
DEPTH = 6
GA = 4
N_CHUNKS = -(-N_EDGES // (NS * CHUNK * DEPTH)) * DEPTH
PER_TILE = N_CHUNKS * CHUNK
E_PAD = PER_TILE * NS
CPT = E_PAD // CHUNK

N_PAD = 51200
ROWS_PER_TILE = N_PAD // NS

_GRP = CHUNK // 16
_ZFULL = ROWS_PER_TILE // CHUNK


def _layer_body(tab, src2, dsth, valh, out, acc, *scratch):
    c = lax.axis_index("c")
    s = lax.axis_index("s")
    sidx = scratch[0:DEPTH]
    vbuf = scratch[DEPTH:2 * DEPTH]
    rows = scratch[2 * DEPTH:3 * DEPTH]
    didx = scratch[3 * DEPTH:4 * DEPTH]
    isem = scratch[4 * DEPTH:5 * DEPTH]
    gsem = scratch[5 * DEPTH:6 * DEPTH]
    ssem = scratch[6 * DEPTH:7 * DEPTH]
    dsem = scratch[7 * DEPTH:8 * DEPTH]

    def _zero(j, carry):
        for b in range(DEPTH):
            rows[b][j, pl.ds(0, 16)] = jnp.zeros((16,), jnp.float32)
            rows[b][j, pl.ds(16, 16)] = jnp.zeros((16,), jnp.float32)
        return carry
    lax.fori_loop(0, CHUNK, _zero, 0, unroll=2)
    slab = s * ROWS_PER_TILE
    for k in range(_ZFULL):
        pltpu.async_copy(rows[k % DEPTH], acc.at[pl.ds(slab + k * CHUNK, CHUNK)],
                         gsem[k % DEPTH])
    for k in range(_ZFULL):
        pltpu.make_async_copy(rows[k % DEPTH],
                              acc.at[pl.ds(slab + k * CHUNK, CHUNK)],
                              gsem[k % DEPTH]).wait()
    plsc.subcore_barrier()

    ebase = c * E_PAD + s * PER_TILE
    base = s * PER_TILE

    def load_meta(i, b):
        pltpu.async_copy(src2.at[pl.ds(ebase + i * CHUNK, CHUNK)], sidx[b],
                         isem[b])
        pltpu.async_copy(valh.at[pl.ds(base + i * CHUNK, CHUNK)], vbuf[b],
                         isem[b])

    def wait_meta(i, b):
        pltpu.make_async_copy(src2.at[pl.ds(ebase + i * CHUNK, CHUNK)],
                              sidx[b], isem[b]).wait()
        pltpu.make_async_copy(valh.at[pl.ds(base + i * CHUNK, CHUNK)],
                              vbuf[b], isem[b]).wait()

    def load_didx(i, b):
        pltpu.async_copy(dsth.at[pl.ds(base + i * CHUNK, CHUNK)], didx[b],
                         dsem[b])

    def wait_didx(i, b):
        pltpu.make_async_copy(dsth.at[pl.ds(base + i * CHUNK, CHUNK)],
                              didx[b], dsem[b]).wait()

    def start_gather(b):
        pltpu.async_copy(tab.at[sidx[b]], rows[b], gsem[b])

    def wait_gather(b):
        pltpu.make_async_copy(tab.at[sidx[b]], rows[b], gsem[b]).wait()

    def wait_scatter(b):
        pltpu.make_async_copy(rows[b], acc.at[didx[b]], ssem[b]).wait()

    def scale(b):
        vb, rb = vbuf[b], rows[b]

        def _grp(g, carry):
            vv = vb[pl.ds(g * 16, 16)]
            for e in range(16):
                bc = vv.at[jnp.full((16,), e, jnp.int32)].get(
                    mode="promise_in_bounds")
                r = g * 16 + e
                rb[r, pl.ds(0, 16)] = rb[r, pl.ds(0, 16)] * bc
                rb[r, pl.ds(16, 16)] = rb[r, pl.ds(16, 16)] * bc
            return carry
        lax.fori_loop(0, _GRP, _grp, 0)

    def step(i, b, *, wait_sc=True, next_gather=True, next_didx=True,
             do_meta=True, wait_dx=True, sync_scatter=False):
        b2 = (b + GA) % DEPTH
        wait_gather(b)
        if wait_sc:
            wait_scatter(b2)
        if next_didx:
            load_didx(i + GA, b2)
        if next_gather:
            wait_meta(i + GA, b2)
            start_gather(b2)
        if wait_dx:
            wait_didx(i, b)
        scale(b)
        if sync_scatter:
            pltpu.sync_copy(rows[b], acc.at[didx[b]], add=True)
        else:
            pltpu.async_copy(rows[b], acc.at[didx[b]], ssem[b], add=True)
        if do_meta:
            load_meta(i + DEPTH, b)

    for b in range(DEPTH):
        load_meta(b, b)
    for b in range(GA):
        load_didx(b, b)
        wait_meta(b, b)
        start_gather(b)
    for i in range(DEPTH):
        step(i, i, wait_sc=(i >= DEPTH - GA))

    def _main(g, carry):
        i0 = DEPTH + g * DEPTH
        for b in range(DEPTH):
            step(i0 + b, b)
        return carry
    lax.fori_loop(0, (N_CHUNKS - 2 * DEPTH) // DEPTH, _main, 0)

    for k in range(DEPTH):
        i = N_CHUNKS - DEPTH + k
        step(i, i % DEPTH,
             next_didx=(k < DEPTH - GA), next_gather=(k < DEPTH - GA),
             do_meta=False, sync_scatter=(k >= GA))

    plsc.subcore_barrier()

    pltpu.sync_copy(acc.at[pl.ds(s * ROWS_PER_TILE, ROWS_PER_TILE)],
                    out.at[pl.ds(c * N_PAD + s * ROWS_PER_TILE, ROWS_PER_TILE)])


_layer = pl.kernel(
    _layer_body,
    out_type=jax.ShapeDtypeStruct((NC * N_PAD, HALF), jnp.float32),
    mesh=plsc.VectorSubcoreMesh(core_axis_name="c", subcore_axis_name="s",
                                num_cores=NC, num_subcores=NS),
    compiler_params=pltpu.CompilerParams(use_tc_tiling_on_sc=False,
                                         needs_layout_passes=False),
    scratch_types=(
        [pltpu.VMEM_SHARED((N_PAD, HALF), jnp.float32)]
        + [pltpu.VMEM((CHUNK,), jnp.int32)] * DEPTH
        + [pltpu.VMEM((CHUNK,), jnp.float32)] * DEPTH
        + [pltpu.VMEM((CHUNK, HALF), jnp.float32)] * DEPTH
        + [pltpu.VMEM((CHUNK,), jnp.int32)] * DEPTH
        + [pltpu.SemaphoreType.DMA] * (4 * DEPTH)
    ),
)


def _mean_body(a0, b0, c0, d0, a1, b1, c1, d1, out):
    out[:, pl.ds(0, HALF)] = (a0[...] + b0[...] + c0[...] + d0[...]) * 0.25
    out[:, pl.ds(HALF, HALF)] = (a1[...] + b1[...] + c1[...] + d1[...]) * 0.25


_MBLK = 2048


def _mean(t0, l1, l2, l3):
    s0 = pl.BlockSpec((_MBLK, HALF), lambda i: (i, 0))
    s1 = pl.BlockSpec((_MBLK, HALF), lambda i: (i + N_PAD // _MBLK, 0))
    return pl.pallas_call(
        _mean_body,
        grid=(N_PAD // _MBLK,),
        in_specs=[s0] * 4 + [s1] * 4,
        out_specs=pl.BlockSpec((_MBLK, LATENT), lambda i: (i, 0)),
        out_shape=jax.ShapeDtypeStruct((N_PAD, LATENT), jnp.float32),
    )(t0, l1, l2, l3, t0, l1, l2, l3)


def kernel(adj_indices, adj_values, user_table, item_table):
    dst = adj_indices[0].astype(jnp.int32)
    src = adj_indices[1].astype(jnp.int32)
    pad = E_PAD - N_EDGES
    src = jnp.pad(src, (0, pad))
    dst = jnp.pad(dst, (0, pad))
    val = jnp.pad(adj_values, (0, pad))
    src2 = jnp.concatenate([src, src + N_PAD])

    zpad = jnp.zeros((N_PAD - N_NODES, HALF), jnp.float32)
    t0 = jnp.concatenate([user_table[:, :HALF], item_table[:, :HALF], zpad,
                          user_table[:, HALF:], item_table[:, HALF:], zpad],
                         axis=0)

    l1 = _layer(t0, src2, dst, val)
    l2 = _layer(l1, src2, dst, val)
    l3 = _layer(l2, src2, dst, val)

    emb = _mean(t0, l1, l2, l3)
    return emb[:N_USERS], emb[N_USERS:N_NODES]


# speedup vs baseline: 1.0303x; 1.0303x over previous
import jax
import jax.numpy as jnp
from jax import lax
from jax.experimental import pallas as pl
from jax.experimental.pallas import tpu as pltpu
from jax.experimental.pallas import tpu_sc as plsc

N_USERS = 25000
N_ITEMS = 25000
N_NODES = N_USERS + N_ITEMS
LATENT = 64
N_EDGES = 800000

NC = 2
NS = 16
HALF = LATENT // NC

CHUNK = 128
DEPTH = 6
GA = 3
N_CHUNKS = -(-N_EDGES // (NS * CHUNK * DEPTH)) * DEPTH
PER_TILE = N_CHUNKS * CHUNK
E_PAD = PER_TILE * NS
CPT = E_PAD // CHUNK

N_PAD = 51200
ROWS_PER_TILE = N_PAD // NS

_GRP = CHUNK // 16
_ZFULL = ROWS_PER_TILE // CHUNK


def _layer_body(tab, src2, dsth, valh, out, acc, *scratch):
    c = lax.axis_index("c")
    s = lax.axis_index("s")
    sidx = scratch[0:DEPTH]
    vbuf = scratch[DEPTH:2 * DEPTH]
    rows = scratch[2 * DEPTH:3 * DEPTH]
    didx = scratch[3 * DEPTH:4 * DEPTH]
    isem = scratch[4 * DEPTH:5 * DEPTH]
    gsem = scratch[5 * DEPTH:6 * DEPTH]
    ssem = scratch[6 * DEPTH:7 * DEPTH]
    dsem = scratch[7 * DEPTH:8 * DEPTH]

    def _zero(j, carry):
        for b in range(DEPTH):
            rows[b][j, pl.ds(0, 16)] = jnp.zeros((16,), jnp.float32)
            rows[b][j, pl.ds(16, 16)] = jnp.zeros((16,), jnp.float32)
        return carry
    lax.fori_loop(0, CHUNK, _zero, 0, unroll=2)
    slab = s * ROWS_PER_TILE
    for k in range(_ZFULL):
        pltpu.async_copy(rows[k % DEPTH], acc.at[pl.ds(slab + k * CHUNK, CHUNK)],
                         gsem[k % DEPTH])
    for k in range(_ZFULL):
        pltpu.make_async_copy(rows[k % DEPTH],
                              acc.at[pl.ds(slab + k * CHUNK, CHUNK)],
                              gsem[k % DEPTH]).wait()
    plsc.subcore_barrier()

    ebase = c * E_PAD + s * PER_TILE
    base = s * PER_TILE

    def load_meta(i, b):
        pltpu.async_copy(src2.at[pl.ds(ebase + i * CHUNK, CHUNK)], sidx[b],
                         isem[b])
        pltpu.async_copy(valh.at[pl.ds(base + i * CHUNK, CHUNK)], vbuf[b],
                         isem[b])

    def wait_meta(i, b):
        pltpu.make_async_copy(src2.at[pl.ds(ebase + i * CHUNK, CHUNK)],
                              sidx[b], isem[b]).wait()
        pltpu.make_async_copy(valh.at[pl.ds(base + i * CHUNK, CHUNK)],
                              vbuf[b], isem[b]).wait()

    def load_didx(i, b):
        pltpu.async_copy(dsth.at[pl.ds(base + i * CHUNK, CHUNK)], didx[b],
                         dsem[b])

    def wait_didx(i, b):
        pltpu.make_async_copy(dsth.at[pl.ds(base + i * CHUNK, CHUNK)],
                              didx[b], dsem[b]).wait()

    def start_gather(b):
        pltpu.async_copy(tab.at[sidx[b]], rows[b], gsem[b])

    def wait_gather(b):
        pltpu.make_async_copy(tab.at[sidx[b]], rows[b], gsem[b]).wait()

    def wait_scatter(b):
        pltpu.make_async_copy(rows[b], acc.at[didx[b]], ssem[b]).wait()

    def scale(b):
        vb, rb = vbuf[b], rows[b]

        def _grp(g, carry):
            vv = vb[pl.ds(g * 16, 16)]
            for e in range(16):
                bc = vv.at[jnp.full((16,), e, jnp.int32)].get(
                    mode="promise_in_bounds")
                r = g * 16 + e
                rb[r, pl.ds(0, 16)] = rb[r, pl.ds(0, 16)] * bc
                rb[r, pl.ds(16, 16)] = rb[r, pl.ds(16, 16)] * bc
            return carry
        lax.fori_loop(0, _GRP, _grp, 0)

    def step(i, b, *, wait_sc=True, next_gather=True, next_didx=True,
             do_meta=True, wait_dx=True, sync_scatter=False):
        b2 = (b + GA) % DEPTH
        wait_gather(b)
        if wait_sc:
            wait_scatter(b2)
        if next_didx:
            load_didx(i + GA, b2)
        if next_gather:
            wait_meta(i + GA, b2)
            start_gather(b2)
        if wait_dx:
            wait_didx(i, b)
        scale(b)
        if sync_scatter:
            pltpu.sync_copy(rows[b], acc.at[didx[b]], add=True)
        else:
            pltpu.async_copy(rows[b], acc.at[didx[b]], ssem[b], add=True)
        if do_meta:
            load_meta(i + DEPTH, b)

    for b in range(DEPTH):
        load_meta(b, b)
    for b in range(GA):
        load_didx(b, b)
        wait_meta(b, b)
        start_gather(b)
    for i in range(DEPTH):
        step(i, i, wait_sc=(i >= DEPTH - GA))

    def _main(g, carry):
        i0 = DEPTH + g * DEPTH
        for b in range(DEPTH):
            step(i0 + b, b)
        return carry
    lax.fori_loop(0, (N_CHUNKS - 2 * DEPTH) // DEPTH, _main, 0)

    for k in range(DEPTH):
        i = N_CHUNKS - DEPTH + k
        step(i, i % DEPTH,
             next_didx=(k < DEPTH - GA), next_gather=(k < DEPTH - GA),
             do_meta=False, sync_scatter=(k >= GA))

    plsc.subcore_barrier()

    pltpu.sync_copy(acc.at[pl.ds(s * ROWS_PER_TILE, ROWS_PER_TILE)],
                    out.at[pl.ds(c * N_PAD + s * ROWS_PER_TILE, ROWS_PER_TILE)])


_layer = pl.kernel(
    _layer_body,
    out_type=jax.ShapeDtypeStruct((NC * N_PAD, HALF), jnp.float32),
    mesh=plsc.VectorSubcoreMesh(core_axis_name="c", subcore_axis_name="s",
                                num_cores=NC, num_subcores=NS),
    compiler_params=pltpu.CompilerParams(use_tc_tiling_on_sc=False,
                                         needs_layout_passes=False),
    scratch_types=(
        [pltpu.VMEM_SHARED((N_PAD, HALF), jnp.float32)]
        + [pltpu.VMEM((CHUNK,), jnp.int32)] * DEPTH
        + [pltpu.VMEM((CHUNK,), jnp.float32)] * DEPTH
        + [pltpu.VMEM((CHUNK, HALF), jnp.float32)] * DEPTH
        + [pltpu.VMEM((CHUNK,), jnp.int32)] * DEPTH
        + [pltpu.SemaphoreType.DMA] * (4 * DEPTH)
    ),
)


def _mean_body(a0, b0, c0, d0, a1, b1, c1, d1, out):
    out[:, pl.ds(0, HALF)] = (a0[...] + b0[...] + c0[...] + d0[...]) * 0.25
    out[:, pl.ds(HALF, HALF)] = (a1[...] + b1[...] + c1[...] + d1[...]) * 0.25


_MBLK = 2048


def _mean(t0, l1, l2, l3):
    s0 = pl.BlockSpec((_MBLK, HALF), lambda i: (i, 0))
    s1 = pl.BlockSpec((_MBLK, HALF), lambda i: (i + N_PAD // _MBLK, 0))
    return pl.pallas_call(
        _mean_body,
        grid=(N_PAD // _MBLK,),
        in_specs=[s0] * 4 + [s1] * 4,
        out_specs=pl.BlockSpec((_MBLK, LATENT), lambda i: (i, 0)),
        out_shape=jax.ShapeDtypeStruct((N_PAD, LATENT), jnp.float32),
    )(t0, l1, l2, l3, t0, l1, l2, l3)


def kernel(adj_indices, adj_values, user_table, item_table):
    dst = adj_indices[0].astype(jnp.int32)
    src = adj_indices[1].astype(jnp.int32)
    pad = E_PAD - N_EDGES
    src = jnp.pad(src, (0, pad))
    dst = jnp.pad(dst, (0, pad))
    val = jnp.pad(adj_values, (0, pad))
    src2 = jnp.concatenate([src, src + N_PAD])

    zpad = jnp.zeros((N_PAD - N_NODES, HALF), jnp.float32)
    t0 = jnp.concatenate([user_table[:, :HALF], item_table[:, :HALF], zpad,
                          user_table[:, HALF:], item_table[:, HALF:], zpad],
                         axis=0)

    l1 = _layer(t0, src2, dst, val)
    l2 = _layer(l1, src2, dst, val)
    l3 = _layer(l2, src2, dst, val)

    emb = _mean(t0, l1, l2, l3)
    return emb[:N_USERS], emb[N_USERS:N_NODES]
